# baseline (speedup 1.0000x reference)
import functools

import jax
import jax.numpy as jnp
from jax import lax
from jax.experimental import pallas as pl
from jax.experimental.pallas import tpu as pltpu
from jax.experimental.pallas import tpu_sc as plsc

N = 10000
E = 320000
D = 128
B = 256
SEQ = 1280

NC = 2
NS = 16
NW = NC * NS
EPW = E // NW
CH = 80
CH2 = 80
NCH2 = EPW // CH2
ACH = 126
EPAD = ACH * CH2
NZCH = N // CH
ZITER = (NZCH + NS - 1) // NS

_mesh = plsc.VectorSubcoreMesh(core_axis_name="c", subcore_axis_name="s",
                               num_cores=NC, num_subcores=NS)


def _fill_zero_rows(ref, nrows, width):
    def body(i, _):
        for j in range(width // 16):
            ref[i, pl.ds(16 * j, 16)] = jnp.zeros((16,), jnp.float32)
        return 0
    lax.fori_loop(0, nrows, body, 0)


def _reg_copy_idx(src_ref, off, dst_ref, n):
    for k in range(n // 16):
        dst_ref[pl.ds(16 * k, 16)] = src_ref[pl.ds(off + 16 * k, 16)]


def _fill_one_rows(ref, nrows, width):
    def body(i, _):
        for j in range(width // 16):
            ref[i, pl.ds(16 * j, 16)] = jnp.ones((16,), jnp.float32)
        return 0
    lax.fori_loop(0, nrows, body, 0)


_SCAT_LAG = 8


@functools.partial(
    pl.kernel,
    out_type=jax.ShapeDtypeStruct((2 * N, 16), jnp.float32),
    mesh=_mesh,
    scratch_types=[
        pltpu.VMEM((EPW,), jnp.int32),
        pltpu.VMEM((CH2,), jnp.int32),
        pltpu.VMEM((CH2,), jnp.int32),
        pltpu.VMEM((CH2, 16), jnp.float32),
        pltpu.VMEM((CH2, 16), jnp.float32),
        pltpu.VMEM_SHARED((N, 16), jnp.float32),
        pltpu.SemaphoreType.DMA,
    ],
)
def _sc_deg(dst_hbm, out_hbm, dstm, dv0, dv1, ones_v, zrow_v, deg_sh, ssem):
    c = lax.axis_index("c")
    s = lax.axis_index("s")
    wid = c * NS + s
    pltpu.sync_copy(dst_hbm.at[wid], dstm)
    _fill_one_rows(ones_v, CH2, 16)
    _fill_zero_rows(zrow_v, CH2, 16)

    def zbody(k, _):
        idx = s + k * NS
        @pl.when(idx < NZCH)
        def _():
            pltpu.sync_copy(zrow_v.at[pl.ds(0, CH)], deg_sh.at[pl.ds(idx * CH, CH)])
        return 0
    lax.fori_loop(0, ZITER, zbody, 0)
    plsc.subcore_barrier()

    def body(j, _):
        _reg_copy_idx(dstm, j * CH2, dv0, CH2)
        pltpu.sync_copy(ones_v, deg_sh.at[dv0], add=True)
        return 0
    lax.fori_loop(0, NCH2, body, 0)
    plsc.subcore_barrier()

    def wbody(k, _):
        idx = s + k * NS
        @pl.when(idx < NZCH)
        def _():
            pltpu.sync_copy(deg_sh.at[pl.ds(idx * CH, CH)],
                            out_hbm.at[pl.ds(c * N + idx * CH, CH)])
        return 0
    lax.fori_loop(0, ZITER, wbody, 0)


@functools.partial(
    pl.kernel,
    out_type=jax.ShapeDtypeStruct((2 * N, D), jnp.float32),
    mesh=_mesh,
    scratch_types=[
        pltpu.VMEM((EPAD,), jnp.int32),
        pltpu.VMEM((CH2,), jnp.int32),
        pltpu.VMEM((CH2,), jnp.int32),
        pltpu.VMEM((CH2,), jnp.int32),
        pltpu.VMEM((CH2,), jnp.int32),
        pltpu.VMEM((CH2,), jnp.int32),
        pltpu.VMEM((CH2,), jnp.int32),
        pltpu.VMEM((CH2, D), jnp.float32),
        pltpu.VMEM((CH2, D), jnp.float32),
        pltpu.VMEM((CH2, D), jnp.float32),
        pltpu.VMEM_SHARED((N + 8, D), jnp.float32),
        pltpu.SemaphoreType.DMA,
        pltpu.SemaphoreType.DMA,
        pltpu.SemaphoreType.DMA,
        pltpu.SemaphoreType.DMA,
        pltpu.SemaphoreType.DMA,
        pltpu.SemaphoreType.DMA,
    ],
)
def _sc_agg(h_hbm, packed_hbm, out_hbm,
            pkm, sv0, sv1, sv2, dv0, dv1, dv2, rows0, rows1, rows2, acc_sh,
            gs0, gs1, gs2, ss0, ss1, ss2):
    c = lax.axis_index("c")
    s = lax.axis_index("s")
    wid = c * NS + s
    pltpu.sync_copy(packed_hbm.at[wid], pkm)
    _fill_zero_rows(rows0, CH2, D)

    def zbody(k, _):
        idx = s + k * NS
        @pl.when(idx < NZCH)
        def _():
            pltpu.sync_copy(rows0.at[pl.ds(0, CH)], acc_sh.at[pl.ds(idx * CH, CH)])
        return 0
    lax.fori_loop(0, ZITER, zbody, 0)
    plsc.subcore_barrier()

    svs = (sv0, sv1, sv2)
    dvs = (dv0, dv1, dv2)
    bufs = (rows0, rows1, rows2)
    gsems = (gs0, gs1, gs2)
    ssems = (ss0, ss1, ss2)

    def unpack(j, bi):
        for g in range(CH2 // 16):
            p = pkm[pl.ds(j * CH2 + 16 * g, 16)]
            svs[bi][pl.ds(16 * g, 16)] = p & 0xFFFF
            dvs[bi][pl.ds(16 * g, 16)] = p >> 16

    def g_issue(j, bi):
        pltpu.async_copy(h_hbm.at[svs[bi]], bufs[bi], gsems[bi])

    def g_wait(bi):
        pltpu.make_async_copy(h_hbm.at[svs[bi]], bufs[bi],
                              gsems[bi]).wait()

    def s_issue(bi):
        pltpu.async_copy(bufs[bi], acc_sh.at[dvs[bi]], ssems[bi], add=True)

    def s_wait(bi):
        pltpu.make_async_copy(bufs[bi], acc_sh.at[dvs[bi]], ssems[bi]).wait()

    unpack(0, 0)
    g_issue(0, 0)
    unpack(1, 1)
    g_issue(1, 1)
    g_wait(0)
    unpack(2, 2)
    g_issue(2, 2)
    s_issue(0)

    def step(k, b):
        bn = (b + 2) % 3
        g_wait(b)
        s_wait(bn)
        unpack_j = k + 2
        unpack(unpack_j, bn)
        g_issue(unpack_j, bn)
        s_issue(b)

    def body(t, _):
        k0 = 3 * t + 1
        step(k0, 1)
        step(k0 + 1, 2)
        step(k0 + 2, 0)
        return 0
    lax.fori_loop(0, (ACH - 3) // 3, body, 0)

    b = (ACH - 2) % 3
    g_wait(b)
    s_wait((b + 2) % 3)
    s_issue(b)
    b = (ACH - 1) % 3
    g_wait(b)
    s_wait((b + 2) % 3)
    s_issue(b)
    s_wait((ACH - 1) % 3)
    plsc.subcore_barrier()

    def wbody(k, _):
        idx = s + k * NS
        @pl.when(idx < NZCH)
        def _():
            pltpu.sync_copy(acc_sh.at[pl.ds(idx * CH, CH)],
                            out_hbm.at[pl.ds(c * N + idx * CH, CH)])
        return 0
    lax.fori_loop(0, ZITER, wbody, 0)


@functools.partial(
    pl.kernel,
    out_type=(jax.ShapeDtypeStruct((2 * B, D), jnp.float32),
              jax.ShapeDtypeStruct((2 * B, 16), jnp.float32)),
    mesh=_mesh,
    scratch_types=[
        pltpu.VMEM((CH,), jnp.int32),
        pltpu.VMEM((CH, D), jnp.float32),
        pltpu.VMEM((CH, 16), jnp.float32),
        pltpu.VMEM((16, D), jnp.float32),
        pltpu.VMEM((16, 16), jnp.float32),
        pltpu.VMEM_SHARED((B, D), jnp.float32),
        pltpu.VMEM_SHARED((B, 16), jnp.float32),
    ],
)
def _sc_pool(h_hbm, batch_hbm, sum_hbm, cnt_hbm,
             bidx_v, rows_v, ones_v, zrow_v, zcnt_v, sum_sh, cnt_sh):
    c = lax.axis_index("c")
    s = lax.axis_index("s")
    wid = c * NS + s
    _fill_one_rows(ones_v, CH, 16)
    _fill_zero_rows(zrow_v, 16, D)
    _fill_zero_rows(zcnt_v, 16, 16)

    pltpu.sync_copy(zrow_v, sum_sh.at[pl.ds(16 * s, 16)])
    pltpu.sync_copy(zcnt_v, cnt_sh.at[pl.ds(16 * s, 16)])
    plsc.subcore_barrier()

    nrch = N // CH
    def body(k, _):
        idx = wid + k * NW
        @pl.when(idx < nrch)
        def _():
            off = idx * CH
            pltpu.sync_copy(batch_hbm.at[pl.ds(off, CH)], bidx_v)
            pltpu.sync_copy(h_hbm.at[pl.ds(off, CH)], rows_v)
            pltpu.sync_copy(rows_v, sum_sh.at[bidx_v], add=True)
            pltpu.sync_copy(ones_v, cnt_sh.at[bidx_v], add=True)
        return 0
    lax.fori_loop(0, (nrch + NW - 1) // NW, body, 0)
    plsc.subcore_barrier()

    pltpu.sync_copy(sum_sh.at[pl.ds(16 * s, 16)],
                    sum_hbm.at[pl.ds(c * B + 16 * s, 16)])
    pltpu.sync_copy(cnt_sh.at[pl.ds(16 * s, 16)],
                    cnt_hbm.at[pl.ds(c * B + 16 * s, 16)])


def _tc_pre_body(x_ref, w_ref, degp_ref, dis_ref, hs_ref):
    dp = degp_ref[...]
    deg = dp[0:N, 0:1] + dp[N:2 * N, 0:1] + 1.0
    dis = lax.rsqrt(deg)
    dis_ref[...] = dis
    h = jnp.dot(x_ref[...], w_ref[...], preferred_element_type=jnp.float32)
    hs_ref[...] = h * dis


_tc_pre = pl.pallas_call(
    _tc_pre_body,
    out_shape=(jax.ShapeDtypeStruct((N, 1), jnp.float32),
               jax.ShapeDtypeStruct((N, D), jnp.float32)),
)


def _bn_block(r, g, be):
    m = jnp.mean(r, axis=0, keepdims=True)
    v = jnp.mean((r - m) * (r - m), axis=0, keepdims=True)
    return g * (r - m) * lax.rsqrt(v + 1e-5) + be


def _tc_mid_body(agg_ref, hs_ref, dis_ref, bc_ref, g_ref, be_ref, w_ref,
                 out_ref):
    a = agg_ref[...]
    dis = dis_ref[...]
    z = dis * (a[0:N, :] + a[N:2 * N, :] + hs_ref[...]) + bc_ref[...]
    r = jnp.maximum(z, 0.0)
    bn = _bn_block(r, g_ref[...], be_ref[...])
    out_ref[...] = dis * jnp.dot(bn, w_ref[...],
                                 preferred_element_type=jnp.float32)


_tc_mid = pl.pallas_call(
    _tc_mid_body,
    out_shape=jax.ShapeDtypeStruct((N, D), jnp.float32),
)


def _tc_last_body(agg_ref, hs_ref, dis_ref, bc_ref, g_ref, be_ref, out_ref):
    a = agg_ref[...]
    dis = dis_ref[...]
    z = dis * (a[0:N, :] + a[N:2 * N, :] + hs_ref[...]) + bc_ref[...]
    out_ref[...] = _bn_block(z, g_ref[...], be_ref[...])


_tc_last = pl.pallas_call(
    _tc_last_body,
    out_shape=jax.ShapeDtypeStruct((N, D), jnp.float32),
)


def _tc_head_body(pp_ref, cc_ref, seq_ref, wfc_ref, bfc_ref,
                  wl0_ref, bl0_ref, wl1_ref, bl1_ref, o1_ref, o2_ref):
    pp = pp_ref[...]
    cc = cc_ref[...]
    cnt = jnp.maximum(cc[0:B, 0:1] + cc[B:2 * B, 0:1], 1.0)
    pooled = (pp[0:B, :] + pp[B:2 * B, :]) / cnt
    p = pooled + jnp.dot(seq_ref[...], wfc_ref[...],
                         preferred_element_type=jnp.float32) + bfc_ref[...]
    a1 = jnp.dot(p, wl0_ref[...], preferred_element_type=jnp.float32) + bl0_ref[...]
    a2 = jnp.dot(p, wl1_ref[...], preferred_element_type=jnp.float32) + bl1_ref[...]
    o1_ref[...] = 1.0 / (1.0 + jnp.exp(-a1))
    o2_ref[...] = 1.0 / (1.0 + jnp.exp(-a2))


_tc_head = pl.pallas_call(
    _tc_head_body,
    out_shape=(jax.ShapeDtypeStruct((B, 183), jnp.float32),
               jax.ShapeDtypeStruct((B, 182), jnp.float32)),
)


def kernel(x, edge_index, batch, seq, W1, bc1, W2, bc2, W3, bc3,
           g1, be1, g2, be2, g3, be3, Wfc, bfc, Wl0, bl0, Wl1, bl1):
    src = edge_index[0].astype(jnp.int32).reshape(NW, EPW)
    dst = edge_index[1].astype(jnp.int32).reshape(NW, EPW)
    pad = jnp.full((NW, EPAD - EPW), N << 16, jnp.int32)
    packed = jnp.concatenate([src | (dst << 16), pad], axis=1)
    batch = batch.astype(jnp.int32)
    bc1r = bc1.reshape(1, D); bc2r = bc2.reshape(1, D); bc3r = bc3.reshape(1, D)
    g1r = g1.reshape(1, D); g2r = g2.reshape(1, D); g3r = g3.reshape(1, D)
    be1r = be1.reshape(1, D); be2r = be2.reshape(1, D); be3r = be3.reshape(1, D)
    bfcr = bfc.reshape(1, D)
    bl0r = bl0.reshape(1, 183)
    bl1r = bl1.reshape(1, 182)

    degp = _sc_deg(dst)
    dis, hs1 = _tc_pre(x, W1, degp)
    agg1 = _sc_agg(hs1, packed)
    hs2 = _tc_mid(agg1, hs1, dis, bc1r, g1r, be1r, W2)
    agg2 = _sc_agg(hs2, packed)
    hs3 = _tc_mid(agg2, hs2, dis, bc2r, g2r, be2r, W3)
    agg3 = _sc_agg(hs3, packed)
    h3f = _tc_last(agg3, hs3, dis, bc3r, g3r, be3r)
    psum, pcnt = _sc_pool(h3f, batch)
    o1, o2 = _tc_head(psum, pcnt, seq, Wfc, bfcr, Wl0, bl0r, Wl1, bl1r)
    return jnp.concatenate([o1, o2, o2], axis=1)


# speedup vs baseline: 1.2627x; 1.2627x over previous
import functools

import jax
import jax.numpy as jnp
from jax import lax
from jax.experimental import pallas as pl
from jax.experimental.pallas import tpu as pltpu
from jax.experimental.pallas import tpu_sc as plsc

N = 10000
E = 320000
D = 128
B = 256
SEQ = 1280

NC = 2
NS = 16
NW = NC * NS
EPW = E // NW
CH = 80
CH2 = 80
NCH2 = EPW // CH2
NZCH = N // CH
ZITER = (NZCH + NS - 1) // NS

_mesh = plsc.VectorSubcoreMesh(core_axis_name="c", subcore_axis_name="s",
                               num_cores=NC, num_subcores=NS)


def _fill_zero_rows(ref, nrows, width):
    def body(i, _):
        for j in range(width // 16):
            ref[i, pl.ds(16 * j, 16)] = jnp.zeros((16,), jnp.float32)
        return 0
    lax.fori_loop(0, nrows, body, 0)


def _reg_copy_idx(src_ref, off, dst_ref, n):
    for k in range(n // 16):
        dst_ref[pl.ds(16 * k, 16)] = src_ref[pl.ds(off + 16 * k, 16)]


def _fill_one_rows(ref, nrows, width):
    def body(i, _):
        for j in range(width // 16):
            ref[i, pl.ds(16 * j, 16)] = jnp.ones((16,), jnp.float32)
        return 0
    lax.fori_loop(0, nrows, body, 0)


_SCAT_LAG = 8


@functools.partial(
    pl.kernel,
    out_type=jax.ShapeDtypeStruct((2 * N, 16), jnp.float32),
    mesh=_mesh,
    scratch_types=[
        pltpu.VMEM((EPW,), jnp.int32),
        pltpu.VMEM((CH2,), jnp.int32),
        pltpu.VMEM((CH2,), jnp.int32),
        pltpu.VMEM((CH2, 16), jnp.float32),
        pltpu.VMEM((CH2, 16), jnp.float32),
        pltpu.VMEM_SHARED((N, 16), jnp.float32),
        pltpu.SemaphoreType.DMA,
        pltpu.SemaphoreType.DMA,
    ],
)
def _sc_deg(dst_hbm, out_hbm, dstm, dv0, dv1, ones_v, zrow_v, deg_sh,
            ssem, ssem2):
    c = lax.axis_index("c")
    s = lax.axis_index("s")
    wid = c * NS + s
    pltpu.sync_copy(dst_hbm.at[wid], dstm)
    _fill_one_rows(ones_v, CH2, 16)
    _fill_zero_rows(zrow_v, CH2, 16)

    def zbody(k, _):
        idx = s + k * NS
        @pl.when(idx < NZCH)
        def _():
            pltpu.sync_copy(zrow_v.at[pl.ds(0, CH)], deg_sh.at[pl.ds(idx * CH, CH)])
        return 0
    lax.fori_loop(0, ZITER, zbody, 0)
    plsc.subcore_barrier()

    def body(t, _):
        j0 = 2 * t
        _reg_copy_idx(dstm, j0 * CH2, dv0, CH2)
        h0 = pltpu.async_copy(ones_v, deg_sh.at[dv0], ssem, add=True)
        _reg_copy_idx(dstm, (j0 + 1) * CH2, dv1, CH2)
        h0.wait()
        h1 = pltpu.async_copy(ones_v, deg_sh.at[dv1], ssem2, add=True)
        h1.wait()
        return 0
    lax.fori_loop(0, NCH2 // 2, body, 0)
    _reg_copy_idx(dstm, (NCH2 - 1) * CH2, dv0, CH2)
    pltpu.sync_copy(ones_v, deg_sh.at[dv0], add=True)
    plsc.subcore_barrier()

    def wbody(k, _):
        idx = s + k * NS
        @pl.when(idx < NZCH)
        def _():
            pltpu.sync_copy(deg_sh.at[pl.ds(idx * CH, CH)],
                            out_hbm.at[pl.ds(c * N + idx * CH, CH)])
        return 0
    lax.fori_loop(0, ZITER, wbody, 0)


@functools.partial(
    pl.kernel,
    out_type=jax.ShapeDtypeStruct((2 * N, D), jnp.float32),
    mesh=_mesh,
    scratch_types=[
        pltpu.VMEM((EPW,), jnp.int32),
        pltpu.VMEM((EPW,), jnp.int32),
        pltpu.VMEM((CH2,), jnp.int32),
        pltpu.VMEM((CH2, D), jnp.float32),
        pltpu.VMEM((CH2, D), jnp.float32),
        pltpu.VMEM((CH2, D), jnp.float32),
        pltpu.VMEM_SHARED((N, D), jnp.float32),
        pltpu.SemaphoreType.DMA,
        pltpu.SemaphoreType.DMA,
        pltpu.SemaphoreType.DMA,
    ],
)
def _sc_agg(h_hbm, src_hbm, dst_hbm, out_hbm,
            srcm, dstm, dv0, rows0, rows1, rows2, acc_sh,
            gsem0, gsem1, gsem2):
    c = lax.axis_index("c")
    s = lax.axis_index("s")
    wid = c * NS + s
    pltpu.sync_copy(src_hbm.at[wid], srcm)
    pltpu.sync_copy(dst_hbm.at[wid], dstm)
    _fill_zero_rows(rows0, CH2, D)

    def zbody(k, _):
        idx = s + k * NS
        @pl.when(idx < NZCH)
        def _():
            pltpu.sync_copy(rows0.at[pl.ds(0, CH)], acc_sh.at[pl.ds(idx * CH, CH)])
        return 0
    lax.fori_loop(0, ZITER, zbody, 0)
    plsc.subcore_barrier()

    bufs = (rows0, rows1, rows2)
    gsems = (gsem0, gsem1, gsem2)

    def g_start(j, bi):
        return pltpu.async_copy(h_hbm.at[srcm.at[pl.ds(j * CH2, CH2)]],
                                bufs[bi], gsems[bi])

    def step(j, k, buf):
        _reg_copy_idx(dstm, j * CH2, dv0, CH2)
        pltpu.sync_copy(buf, acc_sh.at[dv0], add=True)

    U = 5
    def body(t, _):
        j0 = U * t
        hs = [None] * U
        hs[0] = g_start(j0 + 0, 0)
        hs[1] = g_start(j0 + 1, 1)
        for k in range(U):
            hs[k].wait()
            step(j0 + k, k, bufs[k % 3])
            if k + 2 < U:
                hs[k + 2] = g_start(j0 + k + 2, (k + 2) % 3)
        return 0
    lax.fori_loop(0, NCH2 // U, body, 0)
    plsc.subcore_barrier()

    def wbody(k, _):
        idx = s + k * NS
        @pl.when(idx < NZCH)
        def _():
            pltpu.sync_copy(acc_sh.at[pl.ds(idx * CH, CH)],
                            out_hbm.at[pl.ds(c * N + idx * CH, CH)])
        return 0
    lax.fori_loop(0, ZITER, wbody, 0)


@functools.partial(
    pl.kernel,
    out_type=(jax.ShapeDtypeStruct((2 * B, D), jnp.float32),
              jax.ShapeDtypeStruct((2 * B, 16), jnp.float32)),
    mesh=_mesh,
    scratch_types=[
        pltpu.VMEM((CH,), jnp.int32),
        pltpu.VMEM((CH, D), jnp.float32),
        pltpu.VMEM((CH, 16), jnp.float32),
        pltpu.VMEM((16, D), jnp.float32),
        pltpu.VMEM((16, 16), jnp.float32),
        pltpu.VMEM_SHARED((B, D), jnp.float32),
        pltpu.VMEM_SHARED((B, 16), jnp.float32),
    ],
)
def _sc_pool(h_hbm, batch_hbm, sum_hbm, cnt_hbm,
             bidx_v, rows_v, ones_v, zrow_v, zcnt_v, sum_sh, cnt_sh):
    c = lax.axis_index("c")
    s = lax.axis_index("s")
    wid = c * NS + s
    _fill_one_rows(ones_v, CH, 16)
    _fill_zero_rows(zrow_v, 16, D)
    _fill_zero_rows(zcnt_v, 16, 16)

    pltpu.sync_copy(zrow_v, sum_sh.at[pl.ds(16 * s, 16)])
    pltpu.sync_copy(zcnt_v, cnt_sh.at[pl.ds(16 * s, 16)])
    plsc.subcore_barrier()

    nrch = N // CH
    def body(k, _):
        idx = wid + k * NW
        @pl.when(idx < nrch)
        def _():
            off = idx * CH
            pltpu.sync_copy(batch_hbm.at[pl.ds(off, CH)], bidx_v)
            pltpu.sync_copy(h_hbm.at[pl.ds(off, CH)], rows_v)
            pltpu.sync_copy(rows_v, sum_sh.at[bidx_v], add=True)
            pltpu.sync_copy(ones_v, cnt_sh.at[bidx_v], add=True)
        return 0
    lax.fori_loop(0, (nrch + NW - 1) // NW, body, 0)
    plsc.subcore_barrier()

    pltpu.sync_copy(sum_sh.at[pl.ds(16 * s, 16)],
                    sum_hbm.at[pl.ds(c * B + 16 * s, 16)])
    pltpu.sync_copy(cnt_sh.at[pl.ds(16 * s, 16)],
                    cnt_hbm.at[pl.ds(c * B + 16 * s, 16)])


def _tc_pre_body(x_ref, w_ref, degp_ref, dis_ref, hs_ref):
    dp = degp_ref[...]
    deg = dp[0:N, 0:1] + dp[N:2 * N, 0:1] + 1.0
    dis = lax.rsqrt(deg)
    dis_ref[...] = dis
    h = jnp.dot(x_ref[...], w_ref[...], preferred_element_type=jnp.float32)
    hs_ref[...] = h * dis


_tc_pre = pl.pallas_call(
    _tc_pre_body,
    out_shape=(jax.ShapeDtypeStruct((N, 1), jnp.float32),
               jax.ShapeDtypeStruct((N, D), jnp.float32)),
)


def _bn_block(r, g, be):
    m = jnp.mean(r, axis=0, keepdims=True)
    v = jnp.mean((r - m) * (r - m), axis=0, keepdims=True)
    return g * (r - m) * lax.rsqrt(v + 1e-5) + be


def _tc_mid_body(agg_ref, hs_ref, dis_ref, bc_ref, g_ref, be_ref, w_ref,
                 out_ref):
    a = agg_ref[...]
    dis = dis_ref[...]
    z = dis * (a[0:N, :] + a[N:2 * N, :] + hs_ref[...]) + bc_ref[...]
    r = jnp.maximum(z, 0.0)
    bn = _bn_block(r, g_ref[...], be_ref[...])
    out_ref[...] = dis * jnp.dot(bn, w_ref[...],
                                 preferred_element_type=jnp.float32)


_tc_mid = pl.pallas_call(
    _tc_mid_body,
    out_shape=jax.ShapeDtypeStruct((N, D), jnp.float32),
)


def _tc_last_body(agg_ref, hs_ref, dis_ref, bc_ref, g_ref, be_ref, out_ref):
    a = agg_ref[...]
    dis = dis_ref[...]
    z = dis * (a[0:N, :] + a[N:2 * N, :] + hs_ref[...]) + bc_ref[...]
    out_ref[...] = _bn_block(z, g_ref[...], be_ref[...])


_tc_last = pl.pallas_call(
    _tc_last_body,
    out_shape=jax.ShapeDtypeStruct((N, D), jnp.float32),
)


def _tc_head_body(pp_ref, cc_ref, seq_ref, wfc_ref, bfc_ref,
                  wl0_ref, bl0_ref, wl1_ref, bl1_ref, o1_ref, o2_ref):
    pp = pp_ref[...]
    cc = cc_ref[...]
    cnt = jnp.maximum(cc[0:B, 0:1] + cc[B:2 * B, 0:1], 1.0)
    pooled = (pp[0:B, :] + pp[B:2 * B, :]) / cnt
    p = pooled + jnp.dot(seq_ref[...], wfc_ref[...],
                         preferred_element_type=jnp.float32) + bfc_ref[...]
    a1 = jnp.dot(p, wl0_ref[...], preferred_element_type=jnp.float32) + bl0_ref[...]
    a2 = jnp.dot(p, wl1_ref[...], preferred_element_type=jnp.float32) + bl1_ref[...]
    o1_ref[...] = 1.0 / (1.0 + jnp.exp(-a1))
    o2_ref[...] = 1.0 / (1.0 + jnp.exp(-a2))


_tc_head = pl.pallas_call(
    _tc_head_body,
    out_shape=(jax.ShapeDtypeStruct((B, 183), jnp.float32),
               jax.ShapeDtypeStruct((B, 182), jnp.float32)),
)


def kernel(x, edge_index, batch, seq, W1, bc1, W2, bc2, W3, bc3,
           g1, be1, g2, be2, g3, be3, Wfc, bfc, Wl0, bl0, Wl1, bl1):
    src = edge_index[0].astype(jnp.int32).reshape(NW, EPW)
    dst = edge_index[1].astype(jnp.int32).reshape(NW, EPW)
    batch = batch.astype(jnp.int32)
    bc1r = bc1.reshape(1, D); bc2r = bc2.reshape(1, D); bc3r = bc3.reshape(1, D)
    g1r = g1.reshape(1, D); g2r = g2.reshape(1, D); g3r = g3.reshape(1, D)
    be1r = be1.reshape(1, D); be2r = be2.reshape(1, D); be3r = be3.reshape(1, D)
    bfcr = bfc.reshape(1, D)
    bl0r = bl0.reshape(1, 183)
    bl1r = bl1.reshape(1, 182)

    degp = _sc_deg(dst)
    dis, hs1 = _tc_pre(x, W1, degp)
    agg1 = _sc_agg(hs1, src, dst)
    hs2 = _tc_mid(agg1, hs1, dis, bc1r, g1r, be1r, W2)
    agg2 = _sc_agg(hs2, src, dst)
    hs3 = _tc_mid(agg2, hs2, dis, bc2r, g2r, be2r, W3)
    agg3 = _sc_agg(hs3, src, dst)
    h3f = _tc_last(agg3, hs3, dis, bc3r, g3r, be3r)
    psum, pcnt = _sc_pool(h3f, batch)
    o1, o2 = _tc_head(psum, pcnt, seq, Wfc, bfcr, Wl0, bl0r, Wl1, bl1r)
    return jnp.concatenate([o1, o2, o2], axis=1)
